# baseline (speedup 1.0000x reference)
import functools

import jax
import jax.numpy as jnp
from jax import lax
from jax.experimental import pallas as pl
from jax.experimental.pallas import tpu as pltpu
from jax.experimental.pallas import tpu_sc as plsc

_N_LEVELS = 16
_T = 1 << 15
_MASK = _T - 1
_MASK2 = _MASK * 2
_HASH_PRIME = -1640531535
_HPRIME2 = (2 * 2654435761) % (1 << 32)
_NC = 2
_NS = 16
_LANES = 16
_CHUNK = 4096
_VU = 8

_P1, _P2, _P4, _P8 = 1.5, 2.25, 5.0625, 25.62890625


def _encode_body(uv_hbm, tab_hbm, out_hbm,
                 tab_v, ux0_v, uy0_v, e00_v, e10_v, ux1_v, uy1_v, e01_v, e11_v,
                 sin0, sin1, sout0, sout1):
    n = uv_hbm.shape[1]
    half_n = n // _NC
    nchunks = half_n // _CHUNK
    c = lax.axis_index("c")
    s = lax.axis_index("s")
    lvl = s
    pltpu.sync_copy(tab_hbm.at[lvl], tab_v)

    bufs = ((ux0_v, uy0_v, e00_v, e10_v, sin0, sout0),
            (ux1_v, uy1_v, e01_v, e11_v, sin1, sout1))

    lv = jnp.full((_LANES,), lvl, dtype=jnp.int32)
    scale = jnp.full((_LANES,), 16.0, dtype=jnp.float32)
    scale = scale * jnp.where((lv & 1) != 0, _P1, 1.0).astype(jnp.float32)
    scale = scale * jnp.where((lv & 2) != 0, _P2, 1.0).astype(jnp.float32)
    scale = scale * jnp.where((lv & 4) != 0, _P4, 1.0).astype(jnp.float32)
    scale = scale * jnp.where((lv & 8) != 0, _P8, 1.0).astype(jnp.float32)

    base0 = c * half_n
    row0 = 2 * lvl

    def in_copies(p, ci):
        ux_v, uy_v = bufs[p][0], bufs[p][1]
        sem = bufs[p][4]
        base = base0 + ci * _CHUNK
        return (pltpu.make_async_copy(uv_hbm.at[0, pl.ds(base, _CHUNK)], ux_v, sem),
                pltpu.make_async_copy(uv_hbm.at[1, pl.ds(base, _CHUNK)], uy_v, sem))

    def out_copies(p, ci):
        e0_v, e1_v = bufs[p][2], bufs[p][3]
        sem = bufs[p][5]
        base = base0 + ci * _CHUNK
        return (pltpu.make_async_copy(e0_v, out_hbm.at[row0, pl.ds(base, _CHUNK)], sem),
                pltpu.make_async_copy(e1_v, out_hbm.at[row0 + 1, pl.ds(base, _CHUNK)], sem))

    for cp in in_copies(0, 0) + in_copies(1, 1):
        cp.start()

    def chunk_pair(ci2, carry):
        for p in (0, 1):
            ci = ci2 * 2 + p
            ux_v, uy_v, e0_v, e1_v = bufs[p][0], bufs[p][1], bufs[p][2], bufs[p][3]
            for cp in in_copies(p, ci):
                cp.wait()

            @pl.when(ci2 > 0)
            def _wait_prev_out():
                for cp in out_copies(p, ci):
                    cp.wait()

            @plsc.parallel_loop(0, _CHUNK // _LANES, unroll=_VU)
            def vec_body(i):
                off = i * _LANES
                x = ux_v[pl.ds(off, _LANES)]
                y = uy_v[pl.ds(off, _LANES)]
                px = x * scale
                py = y * scale
                xi = px.astype(jnp.int32)
                yi = py.astype(jnp.int32)
                wx = px - xi.astype(jnp.float32)
                wy = py - yi.astype(jnp.float32)
                xi2 = xi * 2
                hy0 = yi * _HPRIME2
                hy1 = hy0 + _HPRIME2
                x12 = xi2 + 2
                i00 = (xi2 ^ hy0) & _MASK2
                i01 = (xi2 ^ hy1) & _MASK2
                i10 = (x12 ^ hy0) & _MASK2
                i11 = (x12 ^ hy1) & _MASK2
                wx0 = 1.0 - wx
                wy0 = 1.0 - wy
                w00 = wx0 * wy0
                w01 = wx0 * wy
                w10 = wx * wy0
                w11 = wx * wy

                def g(a):
                    return plsc.load_gather(tab_v, [a])

                e0 = g(i00) * w00 + g(i01) * w01 + g(i10) * w10 + g(i11) * w11
                e1 = (g(i00 + 1) * w00 + g(i01 + 1) * w01
                      + g(i10 + 1) * w10 + g(i11 + 1) * w11)
                e0_v[pl.ds(off, _LANES)] = e0
                e1_v[pl.ds(off, _LANES)] = e1

            for cp in out_copies(p, ci):
                cp.start()

            @pl.when(ci + 2 < nchunks)
            def _prefetch_next():
                for cp in in_copies(p, ci + 2):
                    cp.start()

        return carry

    lax.fori_loop(0, nchunks // 2, chunk_pair, 0)
    for p in (0, 1):
        for cp in out_copies(p, nchunks - 2 + p):
            cp.wait()


@functools.lru_cache(maxsize=None)
def _make_encode(n):
    mesh = plsc.VectorSubcoreMesh(core_axis_name="c", subcore_axis_name="s")
    return functools.partial(
        pl.kernel,
        mesh=mesh,
        out_type=jax.ShapeDtypeStruct((2 * _N_LEVELS, n), jnp.float32),
        scratch_types=[
            pltpu.VMEM((_T * 2,), jnp.float32),
            pltpu.VMEM((_CHUNK,), jnp.float32),
            pltpu.VMEM((_CHUNK,), jnp.float32),
            pltpu.VMEM((_CHUNK,), jnp.float32),
            pltpu.VMEM((_CHUNK,), jnp.float32),
            pltpu.VMEM((_CHUNK,), jnp.float32),
            pltpu.VMEM((_CHUNK,), jnp.float32),
            pltpu.VMEM((_CHUNK,), jnp.float32),
            pltpu.VMEM((_CHUNK,), jnp.float32),
            pltpu.SemaphoreType.DMA,
            pltpu.SemaphoreType.DMA,
            pltpu.SemaphoreType.DMA,
            pltpu.SemaphoreType.DMA,
        ],
        compiler_params=pltpu.CompilerParams(needs_layout_passes=False),
    )(_encode_body)


def _mlp_body(x_ref, w1t_ref, w2t_ref, w3t_ref, o_ref):
    x = x_ref[...]
    h = lax.dot_general(w1t_ref[...], x, (((1,), (0,)), ((), ())),
                        preferred_element_type=jnp.float32)
    h = jnp.maximum(h, 0.0)
    h = lax.dot_general(w2t_ref[...], h, (((1,), (0,)), ((), ())),
                        preferred_element_type=jnp.float32)
    h = jnp.maximum(h, 0.0)
    o_ref[...] = lax.dot_general(w3t_ref[...], h, (((1,), (0,)), ((), ())),
                                 preferred_element_type=jnp.float32)


_BN = 4096


def kernel(uv_coords, bake, table, W1, W2, W3):
    del bake
    n = uv_coords.shape[0]
    tab = table.reshape(_N_LEVELS, _T * 2)
    enc = _make_encode(n)(uv_coords.T, tab)
    d_in = 2 * _N_LEVELS
    out_t = pl.pallas_call(
        _mlp_body,
        grid=(n // _BN,),
        in_specs=[
            pl.BlockSpec((d_in, _BN), lambda i: (0, i)),
            pl.BlockSpec((64, d_in), lambda i: (0, 0)),
            pl.BlockSpec((64, 64), lambda i: (0, 0)),
            pl.BlockSpec((3, 64), lambda i: (0, 0)),
        ],
        out_specs=pl.BlockSpec((3, _BN), lambda i: (0, i)),
        out_shape=jax.ShapeDtypeStruct((3, n), jnp.float32),
    )(enc, W1.T, W2.T, W3.T)
    return out_t.T.astype(jnp.float32)


# speedup vs baseline: 1.3999x; 1.3999x over previous
import functools

import jax
import jax.numpy as jnp
from jax import lax
from jax.experimental import pallas as pl
from jax.experimental.pallas import tpu as pltpu
from jax.experimental.pallas import tpu_sc as plsc

_N_LEVELS = 16
_T = 1 << 15
_MASK = _T - 1
_HASH_PRIME = -1640531535
_NC = 2
_LANES = 16
_CHUNK = 4096
_VU = 4

_P1, _P2, _P4, _P8 = 1.5, 2.25, 5.0625, 25.62890625


def _encode_body(ux_hbm, uy_hbm, tab_hbm, out_hbm,
                 tab_v, ux0_v, uy0_v, e0_v, ux1_v, uy1_v, e1_v,
                 sin0, sin1, sout0, sout1):
    n = ux_hbm.shape[0]
    half_n = n // _NC
    nchunks = half_n // _CHUNK
    c = lax.axis_index("c")
    s = lax.axis_index("s")
    lvl = s
    pltpu.sync_copy(tab_hbm.at[lvl], tab_v)

    bufs = ((ux0_v, uy0_v, e0_v, sin0, sout0),
            (ux1_v, uy1_v, e1_v, sin1, sout1))

    lv = jnp.full((_LANES,), lvl, dtype=jnp.int32)
    scale = jnp.full((_LANES,), 16.0, dtype=jnp.float32)
    scale = scale * jnp.where((lv & 1) != 0, _P1, 1.0).astype(jnp.float32)
    scale = scale * jnp.where((lv & 2) != 0, _P2, 1.0).astype(jnp.float32)
    scale = scale * jnp.where((lv & 4) != 0, _P4, 1.0).astype(jnp.float32)
    scale = scale * jnp.where((lv & 8) != 0, _P8, 1.0).astype(jnp.float32)

    base0 = c * half_n

    def in_copies(p, ci):
        ux_v, uy_v = bufs[p][0], bufs[p][1]
        sem = bufs[p][3]
        base = base0 + ci * _CHUNK
        return (pltpu.make_async_copy(ux_hbm.at[pl.ds(base, _CHUNK)], ux_v, sem),
                pltpu.make_async_copy(uy_hbm.at[pl.ds(base, _CHUNK)], uy_v, sem))

    def out_copies(p, ci):
        e_v = bufs[p][2]
        sem = bufs[p][4]
        base = base0 + ci * _CHUNK
        return (pltpu.make_async_copy(e_v, out_hbm.at[lvl, pl.ds(base, _CHUNK)], sem),)

    for cp in in_copies(0, 0) + in_copies(1, 1):
        cp.start()

    def encode_16(x, y):
        px = x * scale
        py = y * scale
        xi = px.astype(jnp.int32)
        yi = py.astype(jnp.int32)
        wx = px - xi.astype(jnp.float32)
        wy = py - yi.astype(jnp.float32)
        hy0 = yi * _HASH_PRIME
        hy1 = hy0 + _HASH_PRIME
        x1 = xi + 1
        i00 = (xi ^ hy0) & _MASK
        i01 = (xi ^ hy1) & _MASK
        i10 = (x1 ^ hy0) & _MASK
        i11 = (x1 ^ hy1) & _MASK
        wx0 = 1.0 - wx
        wy0 = 1.0 - wy
        w00 = wx0 * wy0
        w01 = wx0 * wy
        w10 = wx * wy0
        w11 = wx * wy
        g00 = plsc.load_gather(tab_v, [i00])
        g01 = plsc.load_gather(tab_v, [i01])
        g10 = plsc.load_gather(tab_v, [i10])
        g11 = plsc.load_gather(tab_v, [i11])

        def f0(g):
            return plsc.bitcast(g << 16, jnp.float32)

        def f1(g):
            return plsc.bitcast(g & jnp.int32(-65536), jnp.float32)

        e0 = f0(g00) * w00 + f0(g01) * w01 + f0(g10) * w10 + f0(g11) * w11
        e1 = f1(g00) * w00 + f1(g01) * w01 + f1(g10) * w10 + f1(g11) * w11
        return e0, e1

    def chunk_pair(ci2, carry):
        for p in (0, 1):
            ci = ci2 * 2 + p
            ux_v, uy_v, e_v = bufs[p][0], bufs[p][1], bufs[p][2]
            for cp in in_copies(p, ci):
                cp.wait()

            @pl.when(ci2 > 0)
            def _wait_prev_out():
                for cp in out_copies(p, ci):
                    cp.wait()

            @plsc.parallel_loop(0, _CHUNK // _LANES, unroll=_VU)
            def vec_body(i):
                off = i * _LANES
                x = ux_v[pl.ds(off, _LANES)]
                y = uy_v[pl.ds(off, _LANES)]
                e0, e1 = encode_16(x, y)
                pk = plsc.pack(e0, e1, format=plsc.PackFormat.INTERLEAVED)
                e_v[pl.ds(off, _LANES)] = plsc.bitcast(pk, jnp.int32)

            for cp in out_copies(p, ci):
                cp.start()

            @pl.when(ci + 2 < nchunks)
            def _prefetch_next():
                for cp in in_copies(p, ci + 2):
                    cp.start()

        return carry

    lax.fori_loop(0, nchunks // 2, chunk_pair, 0)
    for p in (0, 1):
        for cp in out_copies(p, nchunks - 2 + p):
            cp.wait()


@functools.lru_cache(maxsize=None)
def _make_encode(n):
    mesh = plsc.VectorSubcoreMesh(core_axis_name="c", subcore_axis_name="s")
    return functools.partial(
        pl.kernel,
        mesh=mesh,
        out_type=jax.ShapeDtypeStruct((_N_LEVELS, n), jnp.int32),
        scratch_types=[
            pltpu.VMEM((_T,), jnp.int32),
            pltpu.VMEM((_CHUNK,), jnp.float32),
            pltpu.VMEM((_CHUNK,), jnp.float32),
            pltpu.VMEM((_CHUNK,), jnp.int32),
            pltpu.VMEM((_CHUNK,), jnp.float32),
            pltpu.VMEM((_CHUNK,), jnp.float32),
            pltpu.VMEM((_CHUNK,), jnp.int32),
            pltpu.SemaphoreType.DMA,
            pltpu.SemaphoreType.DMA,
            pltpu.SemaphoreType.DMA,
            pltpu.SemaphoreType.DMA,
        ],
        compiler_params=pltpu.CompilerParams(needs_layout_passes=False),
    )(_encode_body)


def _mlp_body(x_ref, w1ta_ref, w1tb_ref, w2t_ref, w3t_ref, o_ref):
    x = x_ref[...]
    x0 = lax.bitcast_convert_type(x << 16, jnp.float32)
    x1 = lax.bitcast_convert_type(x & jnp.int32(-65536), jnp.float32)
    h = (lax.dot_general(w1ta_ref[...], x0, (((1,), (0,)), ((), ())),
                         preferred_element_type=jnp.float32)
         + lax.dot_general(w1tb_ref[...], x1, (((1,), (0,)), ((), ())),
                           preferred_element_type=jnp.float32))
    h = jnp.maximum(h, 0.0)
    h = lax.dot_general(w2t_ref[...], h, (((1,), (0,)), ((), ())),
                        preferred_element_type=jnp.float32)
    h = jnp.maximum(h, 0.0)
    o_ref[...] = lax.dot_general(w3t_ref[...], h, (((1,), (0,)), ((), ())),
                                 preferred_element_type=jnp.float32)


_BN = 4096


def kernel(uv_coords, bake, table, W1, W2, W3):
    del bake
    n = uv_coords.shape[0]
    ux = uv_coords[:, 0]
    uy = uv_coords[:, 1]
    tabp = lax.bitcast_convert_type(table.astype(jnp.bfloat16), jnp.int32)
    enc = _make_encode(n)(ux, uy, tabp)
    w1t = W1.T
    out_t = pl.pallas_call(
        _mlp_body,
        grid=(n // _BN,),
        in_specs=[
            pl.BlockSpec((_N_LEVELS, _BN), lambda i: (0, i)),
            pl.BlockSpec((64, _N_LEVELS), lambda i: (0, 0)),
            pl.BlockSpec((64, _N_LEVELS), lambda i: (0, 0)),
            pl.BlockSpec((64, 64), lambda i: (0, 0)),
            pl.BlockSpec((3, 64), lambda i: (0, 0)),
        ],
        out_specs=pl.BlockSpec((3, _BN), lambda i: (0, i)),
        out_shape=jax.ShapeDtypeStruct((3, n), jnp.float32),
    )(enc, w1t[:, 0::2], w1t[:, 1::2], W2.T, W3.T)
    return out_t.T.astype(jnp.float32)


# speedup vs baseline: 1.4896x; 1.0641x over previous
import functools

import jax
import jax.numpy as jnp
from jax import lax
from jax.experimental import pallas as pl
from jax.experimental.pallas import tpu as pltpu
from jax.experimental.pallas import tpu_sc as plsc

_N_LEVELS = 16
_T = 1 << 15
_MASK = _T - 1
_HASH_PRIME = -1640531535
_NC = 2
_LANES = 16
_CHUNK = 4096
_VU = 8

_P1, _P2, _P4, _P8 = 1.5, 2.25, 5.0625, 25.62890625


def _encode_body(ux_hbm, uy_hbm, tab_hbm, out_hbm,
                 tab_v, ux0_v, uy0_v, e0_v, ux1_v, uy1_v, e1_v,
                 sin0, sin1, sout0, sout1):
    n = ux_hbm.shape[0]
    half_n = n // _NC
    nchunks = half_n // _CHUNK
    c = lax.axis_index("c")
    s = lax.axis_index("s")
    lvl = s
    pltpu.sync_copy(tab_hbm.at[lvl], tab_v)

    bufs = ((ux0_v, uy0_v, e0_v, sin0, sout0),
            (ux1_v, uy1_v, e1_v, sin1, sout1))

    lv = jnp.full((_LANES,), lvl, dtype=jnp.int32)
    scale = jnp.full((_LANES,), 16.0, dtype=jnp.float32)
    scale = scale * jnp.where((lv & 1) != 0, _P1, 1.0).astype(jnp.float32)
    scale = scale * jnp.where((lv & 2) != 0, _P2, 1.0).astype(jnp.float32)
    scale = scale * jnp.where((lv & 4) != 0, _P4, 1.0).astype(jnp.float32)
    scale = scale * jnp.where((lv & 8) != 0, _P8, 1.0).astype(jnp.float32)

    base0 = c * half_n

    def in_copies(p, ci):
        ux_v, uy_v = bufs[p][0], bufs[p][1]
        sem = bufs[p][3]
        base = base0 + ci * _CHUNK
        return (pltpu.make_async_copy(ux_hbm.at[pl.ds(base, _CHUNK)], ux_v, sem),
                pltpu.make_async_copy(uy_hbm.at[pl.ds(base, _CHUNK)], uy_v, sem))

    def out_copies(p, ci):
        e_v = bufs[p][2]
        sem = bufs[p][4]
        base = base0 + ci * _CHUNK
        return (pltpu.make_async_copy(e_v, out_hbm.at[lvl, pl.ds(base, _CHUNK)], sem),)

    for cp in in_copies(0, 0) + in_copies(1, 1):
        cp.start()

    def encode_16(x, y):
        px = x * scale
        py = y * scale
        xi = px.astype(jnp.int32)
        yi = py.astype(jnp.int32)
        wx = px - xi.astype(jnp.float32)
        wy = py - yi.astype(jnp.float32)
        hy0 = yi * _HASH_PRIME
        hy1 = hy0 + _HASH_PRIME
        x1 = xi + 1
        i00 = (xi ^ hy0) & _MASK
        i01 = (xi ^ hy1) & _MASK
        i10 = (x1 ^ hy0) & _MASK
        i11 = (x1 ^ hy1) & _MASK
        wx0 = 1.0 - wx
        wy0 = 1.0 - wy
        w00 = wx0 * wy0
        w01 = wx0 * wy
        w10 = wx * wy0
        w11 = wx * wy
        g00 = plsc.load_gather(tab_v, [i00])
        g01 = plsc.load_gather(tab_v, [i01])
        g10 = plsc.load_gather(tab_v, [i10])
        g11 = plsc.load_gather(tab_v, [i11])

        def f0(g):
            return plsc.bitcast(g << 16, jnp.float32)

        def f1(g):
            return plsc.bitcast(g & jnp.int32(-65536), jnp.float32)

        e0 = f0(g00) * w00 + f0(g01) * w01 + f0(g10) * w10 + f0(g11) * w11
        e1 = f1(g00) * w00 + f1(g01) * w01 + f1(g10) * w10 + f1(g11) * w11
        return e0, e1

    def chunk_pair(ci2, carry):
        for p in (0, 1):
            ci = ci2 * 2 + p
            ux_v, uy_v, e_v = bufs[p][0], bufs[p][1], bufs[p][2]
            for cp in in_copies(p, ci):
                cp.wait()

            @pl.when(ci2 > 0)
            def _wait_prev_out():
                for cp in out_copies(p, ci):
                    cp.wait()

            @plsc.parallel_loop(0, _CHUNK // _LANES, unroll=_VU)
            def vec_body(i):
                off = i * _LANES
                x = ux_v[pl.ds(off, _LANES)]
                y = uy_v[pl.ds(off, _LANES)]
                e0, e1 = encode_16(x, y)
                pk = plsc.pack(e0, e1, format=plsc.PackFormat.INTERLEAVED)
                e_v[pl.ds(off, _LANES)] = plsc.bitcast(pk, jnp.int32)

            for cp in out_copies(p, ci):
                cp.start()

            @pl.when(ci + 2 < nchunks)
            def _prefetch_next():
                for cp in in_copies(p, ci + 2):
                    cp.start()

        return carry

    lax.fori_loop(0, nchunks // 2, chunk_pair, 0)
    for p in (0, 1):
        for cp in out_copies(p, nchunks - 2 + p):
            cp.wait()


@functools.lru_cache(maxsize=None)
def _make_encode(n):
    mesh = plsc.VectorSubcoreMesh(core_axis_name="c", subcore_axis_name="s")
    return functools.partial(
        pl.kernel,
        mesh=mesh,
        out_type=jax.ShapeDtypeStruct((_N_LEVELS, n), jnp.int32),
        scratch_types=[
            pltpu.VMEM((_T,), jnp.int32),
            pltpu.VMEM((_CHUNK,), jnp.float32),
            pltpu.VMEM((_CHUNK,), jnp.float32),
            pltpu.VMEM((_CHUNK,), jnp.int32),
            pltpu.VMEM((_CHUNK,), jnp.float32),
            pltpu.VMEM((_CHUNK,), jnp.float32),
            pltpu.VMEM((_CHUNK,), jnp.int32),
            pltpu.SemaphoreType.DMA,
            pltpu.SemaphoreType.DMA,
            pltpu.SemaphoreType.DMA,
            pltpu.SemaphoreType.DMA,
        ],
        compiler_params=pltpu.CompilerParams(needs_layout_passes=False),
    )(_encode_body)


def _mlp_body(x_ref, w1ta_ref, w1tb_ref, w2t_ref, w3t_ref, o_ref):
    x = x_ref[...]
    x0 = lax.bitcast_convert_type(x << 16, jnp.float32)
    x1 = lax.bitcast_convert_type(x & jnp.int32(-65536), jnp.float32)
    h = (lax.dot_general(w1ta_ref[...], x0, (((1,), (0,)), ((), ())),
                         preferred_element_type=jnp.float32)
         + lax.dot_general(w1tb_ref[...], x1, (((1,), (0,)), ((), ())),
                           preferred_element_type=jnp.float32))
    h = jnp.maximum(h, 0.0)
    h = lax.dot_general(w2t_ref[...], h, (((1,), (0,)), ((), ())),
                        preferred_element_type=jnp.float32)
    h = jnp.maximum(h, 0.0)
    o_ref[...] = lax.dot_general(w3t_ref[...], h, (((1,), (0,)), ((), ())),
                                 preferred_element_type=jnp.float32)


_BN = 8192


def kernel(uv_coords, bake, table, W1, W2, W3):
    del bake
    n = uv_coords.shape[0]
    ux = uv_coords[:, 0]
    uy = uv_coords[:, 1]
    tabp = lax.bitcast_convert_type(table.astype(jnp.bfloat16), jnp.int32)
    enc = _make_encode(n)(ux, uy, tabp)
    w1t = W1.T
    out_t = pl.pallas_call(
        _mlp_body,
        grid=(n // _BN,),
        in_specs=[
            pl.BlockSpec((_N_LEVELS, _BN), lambda i: (0, i)),
            pl.BlockSpec((64, _N_LEVELS), lambda i: (0, 0)),
            pl.BlockSpec((64, _N_LEVELS), lambda i: (0, 0)),
            pl.BlockSpec((64, 64), lambda i: (0, 0)),
            pl.BlockSpec((3, 64), lambda i: (0, 0)),
        ],
        out_specs=pl.BlockSpec((3, _BN), lambda i: (0, i)),
        out_shape=jax.ShapeDtypeStruct((3, n), jnp.float32),
    )(enc, w1t[:, 0::2], w1t[:, 1::2], W2.T, W3.T)
    return out_t.T.astype(jnp.float32)


# speedup vs baseline: 1.5429x; 1.0358x over previous
import functools

import jax
import jax.numpy as jnp
from jax import lax
from jax.experimental import pallas as pl
from jax.experimental.pallas import tpu as pltpu
from jax.experimental.pallas import tpu_sc as plsc

_N_LEVELS = 16
_T = 1 << 15
_MASK = _T - 1
_HASH_PRIME = -1640531535
_NC = 2
_LANES = 16
_CHUNK = 8192
_VU = 4

_P1, _P2, _P4, _P8 = 1.5, 2.25, 5.0625, 25.62890625


def _encode_body(ux_hbm, uy_hbm, tab_hbm, out_hbm,
                 tab_v, ux0_v, uy0_v, e0_v, ux1_v, uy1_v, e1_v,
                 sin0, sin1, sout0, sout1):
    n = ux_hbm.shape[0]
    half_n = n // _NC
    nchunks = half_n // _CHUNK
    c = lax.axis_index("c")
    s = lax.axis_index("s")
    lvl = s
    pltpu.sync_copy(tab_hbm.at[lvl], tab_v)

    bufs = ((ux0_v, uy0_v, e0_v, sin0, sout0),
            (ux1_v, uy1_v, e1_v, sin1, sout1))

    lv = jnp.full((_LANES,), lvl, dtype=jnp.int32)
    scale = jnp.full((_LANES,), 16.0, dtype=jnp.float32)
    scale = scale * jnp.where((lv & 1) != 0, _P1, 1.0).astype(jnp.float32)
    scale = scale * jnp.where((lv & 2) != 0, _P2, 1.0).astype(jnp.float32)
    scale = scale * jnp.where((lv & 4) != 0, _P4, 1.0).astype(jnp.float32)
    scale = scale * jnp.where((lv & 8) != 0, _P8, 1.0).astype(jnp.float32)

    base0 = c * half_n

    def in_copies(p, ci):
        ux_v, uy_v = bufs[p][0], bufs[p][1]
        sem = bufs[p][3]
        base = base0 + ci * _CHUNK
        return (pltpu.make_async_copy(ux_hbm.at[pl.ds(base, _CHUNK)], ux_v, sem),
                pltpu.make_async_copy(uy_hbm.at[pl.ds(base, _CHUNK)], uy_v, sem))

    def out_copies(p, ci):
        e_v = bufs[p][2]
        sem = bufs[p][4]
        base = base0 + ci * _CHUNK
        return (pltpu.make_async_copy(e_v, out_hbm.at[lvl, pl.ds(base, _CHUNK)], sem),)

    for cp in in_copies(0, 0) + in_copies(1, 1):
        cp.start()

    def encode_16(x, y):
        px = x * scale
        py = y * scale
        xi = px.astype(jnp.int32)
        yi = py.astype(jnp.int32)
        wx = px - xi.astype(jnp.float32)
        wy = py - yi.astype(jnp.float32)
        hy0 = yi * _HASH_PRIME
        hy1 = hy0 + _HASH_PRIME
        x1 = xi + 1
        i00 = (xi ^ hy0) & _MASK
        i01 = (xi ^ hy1) & _MASK
        i10 = (x1 ^ hy0) & _MASK
        i11 = (x1 ^ hy1) & _MASK
        wx0 = 1.0 - wx
        wy0 = 1.0 - wy
        w00 = wx0 * wy0
        w01 = wx0 * wy
        w10 = wx * wy0
        w11 = wx * wy
        g00 = plsc.load_gather(tab_v, [i00])
        g01 = plsc.load_gather(tab_v, [i01])
        g10 = plsc.load_gather(tab_v, [i10])
        g11 = plsc.load_gather(tab_v, [i11])

        def f0(g):
            return plsc.bitcast(g << 16, jnp.float32)

        def f1(g):
            return plsc.bitcast(g & jnp.int32(-65536), jnp.float32)

        e0 = f0(g00) * w00 + f0(g01) * w01 + f0(g10) * w10 + f0(g11) * w11
        e1 = f1(g00) * w00 + f1(g01) * w01 + f1(g10) * w10 + f1(g11) * w11
        return e0, e1

    def chunk_pair(ci2, carry):
        for p in (0, 1):
            ci = ci2 * 2 + p
            ux_v, uy_v, e_v = bufs[p][0], bufs[p][1], bufs[p][2]
            for cp in in_copies(p, ci):
                cp.wait()

            @pl.when(ci2 > 0)
            def _wait_prev_out():
                for cp in out_copies(p, ci):
                    cp.wait()

            @plsc.parallel_loop(0, _CHUNK // _LANES, unroll=_VU)
            def vec_body(i):
                off = i * _LANES
                x = ux_v[pl.ds(off, _LANES)]
                y = uy_v[pl.ds(off, _LANES)]
                e0, e1 = encode_16(x, y)
                pk = plsc.pack(e0, e1, format=plsc.PackFormat.INTERLEAVED)
                e_v[pl.ds(off, _LANES)] = plsc.bitcast(pk, jnp.int32)

            for cp in out_copies(p, ci):
                cp.start()

            @pl.when(ci + 2 < nchunks)
            def _prefetch_next():
                for cp in in_copies(p, ci + 2):
                    cp.start()

        return carry

    lax.fori_loop(0, nchunks // 2, chunk_pair, 0)
    for p in (0, 1):
        for cp in out_copies(p, nchunks - 2 + p):
            cp.wait()


@functools.lru_cache(maxsize=None)
def _make_encode(n):
    mesh = plsc.VectorSubcoreMesh(core_axis_name="c", subcore_axis_name="s")
    return functools.partial(
        pl.kernel,
        mesh=mesh,
        out_type=jax.ShapeDtypeStruct((_N_LEVELS, n), jnp.int32),
        scratch_types=[
            pltpu.VMEM((_T,), jnp.int32),
            pltpu.VMEM((_CHUNK,), jnp.float32),
            pltpu.VMEM((_CHUNK,), jnp.float32),
            pltpu.VMEM((_CHUNK,), jnp.int32),
            pltpu.VMEM((_CHUNK,), jnp.float32),
            pltpu.VMEM((_CHUNK,), jnp.float32),
            pltpu.VMEM((_CHUNK,), jnp.int32),
            pltpu.SemaphoreType.DMA,
            pltpu.SemaphoreType.DMA,
            pltpu.SemaphoreType.DMA,
            pltpu.SemaphoreType.DMA,
        ],
        compiler_params=pltpu.CompilerParams(needs_layout_passes=False),
    )(_encode_body)


def _mlp_body(x_ref, w1ta_ref, w1tb_ref, w2t_ref, w3t_ref, o_ref):
    x = x_ref[...]
    x0 = lax.bitcast_convert_type(x << 16, jnp.float32)
    x1 = lax.bitcast_convert_type(x & jnp.int32(-65536), jnp.float32)
    h = (lax.dot_general(w1ta_ref[...], x0, (((1,), (0,)), ((), ())),
                         preferred_element_type=jnp.float32)
         + lax.dot_general(w1tb_ref[...], x1, (((1,), (0,)), ((), ())),
                           preferred_element_type=jnp.float32))
    h = jnp.maximum(h, 0.0)
    h = lax.dot_general(w2t_ref[...], h, (((1,), (0,)), ((), ())),
                        preferred_element_type=jnp.float32)
    h = jnp.maximum(h, 0.0)
    o_ref[...] = lax.dot_general(w3t_ref[...], h, (((1,), (0,)), ((), ())),
                                 preferred_element_type=jnp.float32)


_BN = 8192


def kernel(uv_coords, bake, table, W1, W2, W3):
    del bake
    n = uv_coords.shape[0]
    ux = uv_coords[:, 0]
    uy = uv_coords[:, 1]
    tabp = lax.bitcast_convert_type(table.astype(jnp.bfloat16), jnp.int32)
    enc = _make_encode(n)(ux, uy, tabp)
    w1t = W1.T
    out_t = pl.pallas_call(
        _mlp_body,
        grid=(n // _BN,),
        in_specs=[
            pl.BlockSpec((_N_LEVELS, _BN), lambda i: (0, i)),
            pl.BlockSpec((64, _N_LEVELS), lambda i: (0, 0)),
            pl.BlockSpec((64, _N_LEVELS), lambda i: (0, 0)),
            pl.BlockSpec((64, 64), lambda i: (0, 0)),
            pl.BlockSpec((3, 64), lambda i: (0, 0)),
        ],
        out_specs=pl.BlockSpec((3, _BN), lambda i: (0, i)),
        out_shape=jax.ShapeDtypeStruct((3, n), jnp.float32),
    )(enc, w1t[:, 0::2], w1t[:, 1::2], W2.T, W3.T)
    return out_t.T.astype(jnp.float32)


# speedup vs baseline: 1.5725x; 1.0192x over previous
import functools

import jax
import jax.numpy as jnp
from jax import lax
from jax.experimental import pallas as pl
from jax.experimental.pallas import tpu as pltpu
from jax.experimental.pallas import tpu_sc as plsc

_N_LEVELS = 16
_T = 1 << 15
_MASK = _T - 1
_HASH_PRIME = -1640531535
_NC = 2
_LANES = 16
_CHUNK = 8192
_VU = 4

_P1, _P2, _P4, _P8 = 1.5, 2.25, 5.0625, 25.62890625


def _encode_body(ux_hbm, uy_hbm, tab_hbm, out_hbm,
                 tab_v, ux0_v, uy0_v, e0_v, ux1_v, uy1_v, e1_v,
                 sin0, sin1, sout0, sout1):
    n = ux_hbm.shape[0]
    half_n = n // _NC
    nchunks = half_n // _CHUNK
    c = lax.axis_index("c")
    s = lax.axis_index("s")
    lvl = s
    pltpu.sync_copy(tab_hbm.at[lvl], tab_v)

    bufs = ((ux0_v, uy0_v, e0_v, sin0, sout0),
            (ux1_v, uy1_v, e1_v, sin1, sout1))

    lv = jnp.full((_LANES,), lvl, dtype=jnp.int32)
    scale = jnp.full((_LANES,), 16.0, dtype=jnp.float32)
    scale = scale * jnp.where((lv & 1) != 0, _P1, 1.0).astype(jnp.float32)
    scale = scale * jnp.where((lv & 2) != 0, _P2, 1.0).astype(jnp.float32)
    scale = scale * jnp.where((lv & 4) != 0, _P4, 1.0).astype(jnp.float32)
    scale = scale * jnp.where((lv & 8) != 0, _P8, 1.0).astype(jnp.float32)

    base0 = c * half_n

    def in_copies(p, ci):
        ux_v, uy_v = bufs[p][0], bufs[p][1]
        sem = bufs[p][3]
        base = base0 + ci * _CHUNK
        return (pltpu.make_async_copy(ux_hbm.at[pl.ds(base, _CHUNK)], ux_v, sem),
                pltpu.make_async_copy(uy_hbm.at[pl.ds(base, _CHUNK)], uy_v, sem))

    def out_copies(p, ci):
        e_v = bufs[p][2]
        sem = bufs[p][4]
        base = base0 + ci * _CHUNK
        return (pltpu.make_async_copy(e_v, out_hbm.at[lvl, pl.ds(base, _CHUNK)], sem),)

    for cp in in_copies(0, 0) + in_copies(1, 1):
        cp.start()

    def encode_16(x, y):
        px = x * scale
        py = y * scale
        xi = px.astype(jnp.int32)
        yi = py.astype(jnp.int32)
        wx = px - xi.astype(jnp.float32)
        wy = py - yi.astype(jnp.float32)
        hy0 = yi * _HASH_PRIME
        hy1 = hy0 + _HASH_PRIME
        x1 = xi + 1
        i00 = (xi ^ hy0) & _MASK
        i01 = (xi ^ hy1) & _MASK
        i10 = (x1 ^ hy0) & _MASK
        i11 = (x1 ^ hy1) & _MASK
        wx0 = 1.0 - wx
        wy0 = 1.0 - wy
        w00 = wx0 * wy0
        w01 = wx0 * wy
        w10 = wx * wy0
        w11 = wx * wy
        g00 = plsc.load_gather(tab_v, [i00])
        g01 = plsc.load_gather(tab_v, [i01])
        g10 = plsc.load_gather(tab_v, [i10])
        g11 = plsc.load_gather(tab_v, [i11])

        def f0(g):
            return plsc.bitcast(g << 16, jnp.float32)

        def f1(g):
            return plsc.bitcast(g & jnp.int32(-65536), jnp.float32)

        e0 = f0(g00) * w00 + f0(g01) * w01 + f0(g10) * w10 + f0(g11) * w11
        e1 = f1(g00) * w00 + f1(g01) * w01 + f1(g10) * w10 + f1(g11) * w11
        return e0, e1

    def chunk_pair(ci2, carry):
        for p in (0, 1):
            ci = ci2 * 2 + p
            ux_v, uy_v, e_v = bufs[p][0], bufs[p][1], bufs[p][2]
            for cp in in_copies(p, ci):
                cp.wait()

            @pl.when(ci2 > 0)
            def _wait_prev_out():
                for cp in out_copies(p, ci):
                    cp.wait()

            @plsc.parallel_loop(0, _CHUNK // _LANES, unroll=_VU)
            def vec_body(i):
                off = i * _LANES
                x = ux_v[pl.ds(off, _LANES)]
                y = uy_v[pl.ds(off, _LANES)]
                e0, e1 = encode_16(x, y)
                pk = plsc.pack(e0, e1, format=plsc.PackFormat.INTERLEAVED)
                e_v[pl.ds(off, _LANES)] = plsc.bitcast(pk, jnp.int32)

            for cp in out_copies(p, ci):
                cp.start()

            @pl.when(ci + 2 < nchunks)
            def _prefetch_next():
                for cp in in_copies(p, ci + 2):
                    cp.start()

        return carry

    lax.fori_loop(0, nchunks // 2, chunk_pair, 0)
    for p in (0, 1):
        for cp in out_copies(p, nchunks - 2 + p):
            cp.wait()


@functools.lru_cache(maxsize=None)
def _make_encode(n):
    mesh = plsc.VectorSubcoreMesh(core_axis_name="c", subcore_axis_name="s")
    return functools.partial(
        pl.kernel,
        mesh=mesh,
        out_type=jax.ShapeDtypeStruct((_N_LEVELS, n), jnp.int32),
        scratch_types=[
            pltpu.VMEM((_T,), jnp.int32),
            pltpu.VMEM((_CHUNK,), jnp.float32),
            pltpu.VMEM((_CHUNK,), jnp.float32),
            pltpu.VMEM((_CHUNK,), jnp.int32),
            pltpu.VMEM((_CHUNK,), jnp.float32),
            pltpu.VMEM((_CHUNK,), jnp.float32),
            pltpu.VMEM((_CHUNK,), jnp.int32),
            pltpu.SemaphoreType.DMA,
            pltpu.SemaphoreType.DMA,
            pltpu.SemaphoreType.DMA,
            pltpu.SemaphoreType.DMA,
        ],
        compiler_params=pltpu.CompilerParams(needs_layout_passes=False),
    )(_encode_body)


def _mlp_body(x_ref, w1ta_ref, w1tb_ref, w2t_ref, w3t_ref, o_ref):
    x = x_ref[...]
    x0 = lax.bitcast_convert_type(x << 16, jnp.float32)
    x1 = lax.bitcast_convert_type(x & jnp.int32(-65536), jnp.float32)
    h = (lax.dot_general(w1ta_ref[...], x0, (((1,), (0,)), ((), ())),
                         preferred_element_type=jnp.float32)
         + lax.dot_general(w1tb_ref[...], x1, (((1,), (0,)), ((), ())),
                           preferred_element_type=jnp.float32))
    h = jnp.maximum(h, 0.0)
    h = lax.dot_general(w2t_ref[...], h, (((1,), (0,)), ((), ())),
                        preferred_element_type=jnp.float32)
    h = jnp.maximum(h, 0.0)
    o_ref[...] = lax.dot_general(w3t_ref[...], h, (((1,), (0,)), ((), ())),
                                 preferred_element_type=jnp.float32)


_BN = 16384


def kernel(uv_coords, bake, table, W1, W2, W3):
    del bake
    n = uv_coords.shape[0]
    ux = uv_coords[:, 0]
    uy = uv_coords[:, 1]
    tabp = lax.bitcast_convert_type(table.astype(jnp.bfloat16), jnp.int32)
    enc = _make_encode(n)(ux, uy, tabp)
    w1t = W1.T
    out_t = pl.pallas_call(
        _mlp_body,
        grid=(n // _BN,),
        in_specs=[
            pl.BlockSpec((_N_LEVELS, _BN), lambda i: (0, i)),
            pl.BlockSpec((64, _N_LEVELS), lambda i: (0, 0)),
            pl.BlockSpec((64, _N_LEVELS), lambda i: (0, 0)),
            pl.BlockSpec((64, 64), lambda i: (0, 0)),
            pl.BlockSpec((3, 64), lambda i: (0, 0)),
        ],
        out_specs=pl.BlockSpec((3, _BN), lambda i: (0, i)),
        out_shape=jax.ShapeDtypeStruct((3, n), jnp.float32),
    )(enc, w1t[:, 0::2], w1t[:, 1::2], W2.T, W3.T)
    return out_t.T.astype(jnp.float32)
